# baseline (speedup 1.0000x reference)
import functools

import jax
import jax.numpy as jnp
from jax import lax
from jax.experimental import pallas as pl
from jax.experimental.pallas import tpu as pltpu
from jax.experimental.pallas import tpu_sc as plsc

NUM_CORES = 2
NUM_SUBCORES = 16
LANES = 16
NW = NUM_CORES * NUM_SUBCORES

BATCH = 16384
B_PER_W = BATCH // NW
CHUNK = 128
NCHUNK = B_PER_W // CHUNK
D = 16


def _mf_body(u_hbm, i_hbm, ut_hbm, it_hbm, out_hbm, *scratch):
    idx_bufs = scratch[0:2 * NCHUNK]
    row_bufs = scratch[2 * NCHUNK:4 * NCHUNK]
    prod_v = scratch[4 * NCHUNK]
    out_v = scratch[4 * NCHUNK + 1]
    sem = scratch[4 * NCHUNK + 2]

    cid = lax.axis_index("c")
    sid = lax.axis_index("s")
    wid = sid * NUM_CORES + cid
    base = wid * B_PER_W

    for j in range(NCHUNK):
        pltpu.sync_copy(u_hbm.at[pl.ds(base + j * CHUNK, CHUNK)], idx_bufs[j])
        pltpu.sync_copy(i_hbm.at[pl.ds(base + j * CHUNK, CHUNK)],
                        idx_bufs[NCHUNK + j])

    copies = []
    for j in range(NCHUNK):
        copies.append(pltpu.async_copy(ut_hbm.at[idx_bufs[j]], row_bufs[j], sem))
        copies.append(pltpu.async_copy(it_hbm.at[idx_bufs[NCHUNK + j]],
                                       row_bufs[NCHUNK + j], sem))
    for c in copies:
        c.wait()

    lane = lax.iota(jnp.int32, LANES)
    for j in range(NCHUNK):
        u_rows = row_bufs[j]
        i_rows = row_bufs[NCHUNK + j]

        def block(b, _, u_rows=u_rows, i_rows=i_rows, j=j):
            acc = jnp.zeros((LANES,), jnp.float32)
            for k in range(LANES):
                r = b * LANES + k
                s = jnp.sum(u_rows[r, :] * i_rows[r, :])
                acc = jnp.where(lane == k, s, acc)
            pred = 1.0 / (1.0 + jnp.exp(-acc))
            out_v[pl.ds(j * CHUNK + b * LANES, LANES)] = pred
            return _

        lax.fori_loop(0, CHUNK // LANES, block, 0)

    pltpu.sync_copy(out_v, out_hbm.at[pl.ds(base, B_PER_W)])


@jax.jit
def _mf(u, i, user_table, item_table):
    mesh = plsc.VectorSubcoreMesh(core_axis_name="c", subcore_axis_name="s")
    scratch = (
        [pltpu.VMEM((CHUNK,), jnp.int32) for _ in range(2 * NCHUNK)]
        + [pltpu.VMEM((CHUNK, D), jnp.float32) for _ in range(2 * NCHUNK)]
        + [pltpu.VMEM((CHUNK * D,), jnp.float32),
           pltpu.VMEM((B_PER_W,), jnp.float32), pltpu.SemaphoreType.DMA]
    )
    run = pl.kernel(
        _mf_body,
        out_type=jax.ShapeDtypeStruct((BATCH,), jnp.float32),
        mesh=mesh,
        scratch_types=scratch,
        compiler_params=pltpu.CompilerParams(
            needs_layout_passes=False, use_tc_tiling_on_sc=False),
    )
    return run(u, i, user_table, item_table)


def kernel(u, i, user_table, item_table):
    return _mf(u.astype(jnp.int32), i.astype(jnp.int32), user_table, item_table)


# speedup vs baseline: 1.4909x; 1.4909x over previous
import functools

import jax
import jax.numpy as jnp
from jax import lax
from jax.experimental import pallas as pl
from jax.experimental.pallas import tpu as pltpu
from jax.experimental.pallas import tpu_sc as plsc

NUM_CORES = 2
NUM_SUBCORES = 16
LANES = 16
NW = NUM_CORES * NUM_SUBCORES

BATCH = 16384
B_PER_W = BATCH // NW
CHUNK = 256
D = 16


def _mf_body(u_hbm, i_hbm, ut_hbm, it_hbm, out_hbm,
             idx_u, idx_i, rows_u, rows_i, out_v, sem_u, sem_i):
    cid = lax.axis_index("c")
    sid = lax.axis_index("s")
    wid = sid * NUM_CORES + cid
    base = wid * B_PER_W

    pltpu.sync_copy(u_hbm.at[pl.ds(base, B_PER_W)], idx_u)
    pltpu.sync_copy(i_hbm.at[pl.ds(base, B_PER_W)], idx_i)

    lane = lax.iota(jnp.int32, LANES)

    for c in range(B_PER_W // CHUNK):
        cbase = c * CHUNK

        def fire(g, _, cbase=cbase):
            vu = idx_u[pl.ds(cbase + g * LANES, LANES)]
            vi = idx_i[pl.ds(cbase + g * LANES, LANES)]
            for k in range(LANES):
                r = g * LANES + k
                pltpu.async_copy(ut_hbm.at[pl.ds(vu[k], 1)],
                                 rows_u.at[pl.ds(r, 1)], sem_u)
                pltpu.async_copy(it_hbm.at[pl.ds(vi[k], 1)],
                                 rows_i.at[pl.ds(r, 1)], sem_i)
            return _

        lax.fori_loop(0, CHUNK // LANES, fire, 0)

        pltpu.make_async_copy(ut_hbm.at[pl.ds(0, CHUNK)], rows_u, sem_u).wait()
        pltpu.make_async_copy(it_hbm.at[pl.ds(0, CHUNK)], rows_i, sem_i).wait()

        def block(b, _, cbase=cbase):
            acc = jnp.zeros((LANES,), jnp.float32)
            for k in range(LANES):
                r = b * LANES + k
                s = jnp.sum(rows_u[r, :] * rows_i[r, :])
                acc = jnp.where(lane == k, s, acc)
            pred = 1.0 / (1.0 + jnp.exp(-acc))
            out_v[pl.ds(cbase + b * LANES, LANES)] = pred
            return _

        lax.fori_loop(0, CHUNK // LANES, block, 0)

    pltpu.sync_copy(out_v, out_hbm.at[pl.ds(base, B_PER_W)])


@jax.jit
def _mf(u, i, user_table, item_table):
    mesh = plsc.VectorSubcoreMesh(core_axis_name="c", subcore_axis_name="s")
    scratch = [
        pltpu.VMEM((B_PER_W,), jnp.int32),
        pltpu.VMEM((B_PER_W,), jnp.int32),
        pltpu.VMEM((CHUNK, D), jnp.float32),
        pltpu.VMEM((CHUNK, D), jnp.float32),
        pltpu.VMEM((B_PER_W,), jnp.float32),
        pltpu.SemaphoreType.DMA,
        pltpu.SemaphoreType.DMA,
    ]
    run = pl.kernel(
        _mf_body,
        out_type=jax.ShapeDtypeStruct((BATCH,), jnp.float32),
        mesh=mesh,
        scratch_types=scratch,
        compiler_params=pltpu.CompilerParams(needs_layout_passes=False),
    )
    return run(u, i, user_table, item_table)


def kernel(u, i, user_table, item_table):
    return _mf(u.astype(jnp.int32), i.astype(jnp.int32), user_table, item_table)
